# baseline (speedup 1.0000x reference)
import jax
import jax.numpy as jnp
from jax import lax
from jax.experimental import pallas as pl

B, N, T, P, D, O = 16, 2048, 1, 6, 128, 128
BLK = 512


def _tc_body(embs_ref, vis_ref, w_ref, b_ref, mask_ref, out_ref):
    e = embs_ref[0, :, 0]
    v = vis_ref[0, :, 0]
    combined = e[:, 0, :]
    for p in range(1, P):
        combined += v[:, p][:, None] * e[:, p, :]
    s = 1.0 + jnp.sum(v[:, 1:], axis=1, keepdims=True)
    y = lax.dot_general(combined, w_ref[...], (((1,), (1,)), ((), ())),
                        preferred_element_type=jnp.float32)
    y = y + s * b_ref[...]
    m = mask_ref[0]
    out_ref[0] = jnp.where(m > 0, y, 0.0)


@jax.jit
def kernel(embs, vis, W, b, masks):
    maskf = masks.astype(jnp.float32)
    b2 = b.reshape(1, O)
    grid = (B, N // BLK)
    out = pl.pallas_call(
        _tc_body,
        grid=grid,
        in_specs=[
            pl.BlockSpec((1, BLK, 1, P, D), lambda i, j: (i, j, 0, 0, 0)),
            pl.BlockSpec((1, BLK, 1, P), lambda i, j: (i, j, 0, 0)),
            pl.BlockSpec((O, D), lambda i, j: (0, 0)),
            pl.BlockSpec((1, O), lambda i, j: (0, 0)),
            pl.BlockSpec((1, BLK, 1), lambda i, j: (i, j, 0)),
        ],
        out_specs=pl.BlockSpec((1, BLK, O), lambda i, j: (i, j, 0)),
        out_shape=jax.ShapeDtypeStruct((B, N, O), jnp.float32),
    )(embs, vis, W, b2, maskf)
    return out


# speedup vs baseline: 1.9956x; 1.9956x over previous
import jax
import jax.numpy as jnp
from jax import lax
from jax.experimental import pallas as pl
from jax.experimental.pallas import tpu as pltpu

B, N, T, P, D, O = 16, 2048, 1, 6, 128, 128
BN = B * N
BLK = 512
NSTEPS = BN // BLK


def _tc_body(embs_hbm, aux_ref, w_ref, b_ref, out_ref, ebuf, sems):
    i = pl.program_id(0)

    def start(step, slot):
        for p in range(P):
            pltpu.make_async_copy(
                embs_hbm.at[pl.ds(step * BLK, BLK), p],
                ebuf.at[slot, p],
                sems.at[slot, p],
            ).start()

    @pl.when(i == 0)
    def _():
        start(0, 0)

    @pl.when(i + 1 < NSTEPS)
    def _():
        start(i + 1, (i + 1) % 2)

    slot = i % 2
    for p in range(P):
        pltpu.make_async_copy(
            embs_hbm.at[pl.ds(i * BLK, BLK), p],
            ebuf.at[slot, p],
            sems.at[slot, p],
        ).wait()

    aux = aux_ref[...].T
    acc = ebuf[slot, 0]
    for p in range(1, P):
        acc += aux[:, p - 1][:, None] * ebuf[slot, p]
    y = lax.dot_general(acc, w_ref[...], (((1,), (1,)), ((), ())),
                        preferred_element_type=jnp.float32)
    y = y + aux[:, 5][:, None] * b_ref[...]
    out_ref[...] = jnp.where(aux[:, 6][:, None] > 0, y, 0.0)


@jax.jit
def kernel(embs, vis, W, b, masks):
    embs3 = embs.reshape(BN, P, D)
    visr = vis.reshape(BN, P)
    c = visr[:, 1:].T
    s = 1.0 + jnp.sum(visr[:, 1:], axis=1)[None, :]
    m = masks.reshape(1, BN).astype(jnp.float32)
    aux = jnp.concatenate([c, s, m, jnp.ones((1, BN), jnp.float32)], axis=0)
    b2 = b.reshape(1, O)
    out = pl.pallas_call(
        _tc_body,
        grid=(NSTEPS,),
        in_specs=[
            pl.BlockSpec(memory_space=pl.ANY),
            pl.BlockSpec((8, BLK), lambda i: (0, i)),
            pl.BlockSpec((O, D), lambda i: (0, 0)),
            pl.BlockSpec((1, O), lambda i: (0, 0)),
        ],
        out_specs=pl.BlockSpec((BLK, O), lambda i: (i, 0)),
        out_shape=jax.ShapeDtypeStruct((BN, O), jnp.float32),
        scratch_shapes=[
            pltpu.VMEM((2, P, BLK, D), jnp.float32),
            pltpu.SemaphoreType.DMA((2, P)),
        ],
    )(embs3, aux, W, b2)
    return out.reshape(B, N, O)


# speedup vs baseline: 2.3225x; 1.1638x over previous
import jax
import jax.numpy as jnp
from jax import lax
from jax.experimental import pallas as pl
from jax.experimental.pallas import tpu as pltpu

B, N, T, P, D, O = 16, 2048, 1, 6, 128, 128
BN = B * N
BLK = 512
NPB = N // BLK
NSTEPS = BN // BLK


def _tc_body(embs_hbm, aux_ref, w_ref, b_ref, out_ref, ebuf, sems):
    i = pl.program_id(0)

    def start(step, slot):
        bi = step // NPB
        j = step % NPB
        for p in range(P):
            pltpu.make_async_copy(
                embs_hbm.at[bi, pl.ds(j * BLK, BLK), 0, p],
                ebuf.at[slot, p],
                sems.at[slot, p],
            ).start()

    def wait(step, slot):
        bi = step // NPB
        j = step % NPB
        for p in range(P):
            pltpu.make_async_copy(
                embs_hbm.at[bi, pl.ds(j * BLK, BLK), 0, p],
                ebuf.at[slot, p],
                sems.at[slot, p],
            ).wait()

    @pl.when(i == 0)
    def _():
        start(0, 0)

    @pl.when(i + 1 < NSTEPS)
    def _():
        start(i + 1, (i + 1) % 2)

    slot = i % 2
    wait(i, slot)

    aux = aux_ref[...].T
    acc = ebuf[slot, 0]
    for p in range(1, P):
        acc += aux[:, p - 1][:, None] * ebuf[slot, p]
    y = lax.dot_general(acc, w_ref[...], (((1,), (1,)), ((), ())),
                        preferred_element_type=jnp.float32)
    y = y + aux[:, 5][:, None] * b_ref[...]
    out_ref[0] = jnp.where(aux[:, 6][:, None] > 0, y, 0.0)


@jax.jit
def kernel(embs, vis, W, b, masks):
    visr = vis.reshape(BN, P)
    c = visr[:, 1:].T
    s = 1.0 + jnp.sum(visr[:, 1:], axis=1)[None, :]
    m = masks.reshape(1, BN).astype(jnp.float32)
    aux = jnp.concatenate([c, s, m, jnp.ones((1, BN), jnp.float32)], axis=0)
    b2 = b.reshape(1, O)
    out = pl.pallas_call(
        _tc_body,
        grid=(NSTEPS,),
        in_specs=[
            pl.BlockSpec(memory_space=pl.ANY),
            pl.BlockSpec((8, BLK), lambda i: (0, i)),
            pl.BlockSpec((O, D), lambda i: (0, 0)),
            pl.BlockSpec((1, O), lambda i: (0, 0)),
        ],
        out_specs=pl.BlockSpec((1, BLK, O), lambda i: (i // NPB, i % NPB, 0)),
        out_shape=jax.ShapeDtypeStruct((B, N, O), jnp.float32),
        scratch_shapes=[
            pltpu.VMEM((2, P, BLK, D), jnp.float32),
            pltpu.SemaphoreType.DMA((2, P)),
        ],
    )(embs, aux, W, b2)
    return out
